# Initial kernel scaffold; baseline (speedup 1.0000x reference)
import jax
import jax.numpy as jnp
from jax.experimental import pallas as pl


def kernel(x_author, x_paper, edge_index_p2a, edge_index_a2p, lin_W, lin_b, Wk, bk, Wq, bq, Wv, bv, Wa, ba, skip, a_rel, m_rel, p_rel):
    raise NotImplementedError("write your pallas kernel here")


# speedup vs baseline: 1.0521x; 1.0521x over previous
import functools
import numpy as np
import jax
import jax.numpy as jnp
from jax.experimental import pallas as pl

NA = 50000
NP_ = 50000
E = 500000
C = 128
H = 4
D = 32
L = 2

BLK = 400


def _matmul_block(x_ref, w_ref, b_ref, o_ref, *, activation):
    acc = jnp.dot(x_ref[...], w_ref[...], preferred_element_type=jnp.float32)
    acc = acc + b_ref[...]
    if activation == "relu":
        acc = jnp.maximum(acc, 0.0)
    o_ref[...] = acc


def _dense_proj(x, w, b, activation="none"):
    n = x.shape[0]
    grid = (n // BLK,)
    return pl.pallas_call(
        functools.partial(_matmul_block, activation=activation),
        grid=grid,
        in_specs=[
            pl.BlockSpec((BLK, C), lambda i: (i, 0)),
            pl.BlockSpec((C, C), lambda i: (0, 0)),
            pl.BlockSpec((1, C), lambda i: (0, 0)),
        ],
        out_specs=pl.BlockSpec((BLK, C), lambda i: (i, 0)),
        out_shape=jax.ShapeDtypeStruct((n, C), jnp.float32),
    )(x, w, b.reshape(1, C))


def kernel(x_author, x_paper, edge_index_p2a, edge_index_a2p, lin_W, lin_b,
           Wk, bk, Wq, bq, Wv, bv, Wa, ba, skip, a_rel, m_rel, p_rel):
    xs = [_dense_proj(x_author, lin_W[0], lin_b[0], "relu"),
          _dense_proj(x_paper, lin_W[1], lin_b[1], "relu")]
    Ns = [NA, NP_]
    edge_types = [(1, 0, edge_index_p2a), (0, 1, edge_index_a2p)]
    for l in range(L):
        k = [_dense_proj(xs[t], Wk[l, t], bk[l, t]).reshape(Ns[t], H, D)
             for t in range(2)]
        q = [_dense_proj(xs[t], Wq[l, t], bq[l, t]).reshape(Ns[t], H, D)
             for t in range(2)]
        v = [_dense_proj(xs[t], Wv[l, t], bv[l, t]).reshape(Ns[t], H, D)
             for t in range(2)]
        agg = [jnp.zeros((Ns[t], H, D), dtype=jnp.float32) for t in range(2)]
        for e in range(2):
            st, dt, ei = edge_types[e]
            ke = jnp.einsum('nhd,hde->nhe', k[st], a_rel[l, e])
            ve = jnp.einsum('nhd,hde->nhe', v[st], m_rel[l, e])
            src = ei[0]
            dst = ei[1]
            logits = (q[dt][dst] * ke[src]).sum(-1) * p_rel[l, e] / np.sqrt(D)
            ex = jnp.exp(logits)
            denom = jax.ops.segment_sum(ex, dst, num_segments=Ns[dt])
            msg = ve[src] * ex[:, :, None]
            unnorm = jax.ops.segment_sum(msg, dst, num_segments=Ns[dt])
            agg[dt] = agg[dt] + unnorm / (denom + 1e-16)[:, :, None]
        new_xs = []
        for t in range(2):
            g = jax.nn.gelu(agg[t].reshape(Ns[t], C))
            out = _dense_proj(g, Wa[l, t], ba[l, t])
            s = jax.nn.sigmoid(skip[l, t])
            new_xs.append(s * out + (1.0 - s) * xs[t])
        xs = new_xs
    return (xs[0], xs[1])


# speedup vs baseline: 13.2624x; 12.6061x over previous
import functools
import numpy as np
import jax
import jax.numpy as jnp
from jax.experimental import pallas as pl

NA = 50000
NP_ = 50000
E = 500000
C = 128
H = 4
D = 32
L = 2

BLK = 2000
BLKE = 4000


def _matmul_block(x_ref, w_ref, b_ref, o_ref, *, activation):
    x = x_ref[...]
    acc = jnp.dot(x, w_ref[...], preferred_element_type=jnp.float32)
    acc = acc + b_ref[...]
    if activation == "relu":
        acc = jnp.maximum(acc, 0.0)
    o_ref[...] = acc


def _dense_proj(x, w, b, activation="none"):
    n, cin = x.shape
    cout = w.shape[1]
    return pl.pallas_call(
        functools.partial(_matmul_block, activation=activation),
        grid=(n // BLK,),
        in_specs=[
            pl.BlockSpec((BLK, cin), lambda i: (i, 0)),
            pl.BlockSpec((cin, cout), lambda i: (0, 0)),
            pl.BlockSpec((1, cout), lambda i: (0, 0)),
        ],
        out_specs=pl.BlockSpec((BLK, cout), lambda i: (i, 0)),
        out_shape=jax.ShapeDtypeStruct((n, cout), jnp.float32),
    )(x, w, b.reshape(1, cout))


def _edge_block(qd_ref, kv_ref, sel_ref, selt_ref, ex_ref, msg_ref):
    qd = qd_ref[...]
    kv = kv_ref[...]
    ke = kv[:, :C]
    ve = kv[:, C:]
    ex = jnp.exp(jnp.dot(qd * ke, sel_ref[...],
                         preferred_element_type=jnp.float32))
    ex_ref[...] = ex
    msg_ref[...] = ve * jnp.dot(ex, selt_ref[...],
                                preferred_element_type=jnp.float32)


def _edge_phase(qd, kv, sel, selt):
    e = qd.shape[0]
    return pl.pallas_call(
        _edge_block,
        grid=(e // BLKE,),
        in_specs=[
            pl.BlockSpec((BLKE, C), lambda i: (i, 0)),
            pl.BlockSpec((BLKE, 2 * C), lambda i: (i, 0)),
            pl.BlockSpec((C, H), lambda i: (0, 0)),
            pl.BlockSpec((H, C), lambda i: (0, 0)),
        ],
        out_specs=[
            pl.BlockSpec((BLKE, H), lambda i: (i, 0)),
            pl.BlockSpec((BLKE, C), lambda i: (i, 0)),
        ],
        out_shape=[
            jax.ShapeDtypeStruct((e, H), jnp.float32),
            jax.ShapeDtypeStruct((e, C), jnp.float32),
        ],
    )(qd, kv, sel, selt)


def _out_block(g_ref, w_ref, b_ref, x_ref, c_ref, o_ref):
    acc = jnp.dot(jax.nn.gelu(g_ref[...]), w_ref[...],
                  preferred_element_type=jnp.float32)
    o_ref[...] = acc + b_ref[...] + c_ref[...] * x_ref[...]


def _out_proj(g, w, b, x_old, c):
    n = g.shape[0]
    return pl.pallas_call(
        _out_block,
        grid=(n // BLK,),
        in_specs=[
            pl.BlockSpec((BLK, C), lambda i: (i, 0)),
            pl.BlockSpec((C, C), lambda i: (0, 0)),
            pl.BlockSpec((1, C), lambda i: (0, 0)),
            pl.BlockSpec((BLK, C), lambda i: (i, 0)),
            pl.BlockSpec((1, C), lambda i: (0, 0)),
        ],
        out_specs=pl.BlockSpec((BLK, C), lambda i: (i, 0)),
        out_shape=jax.ShapeDtypeStruct((n, C), jnp.float32),
    )(g, w, b.reshape(1, C), x_old, c)


def kernel(x_author, x_paper, edge_index_p2a, edge_index_a2p, lin_W, lin_b,
           Wk, bk, Wq, bq, Wv, bv, Wa, ba, skip, a_rel, m_rel, p_rel):
    xs = [_dense_proj(x_author, lin_W[0], lin_b[0], "relu"),
          _dense_proj(x_paper, lin_W[1], lin_b[1], "relu")]
    Ns = [NA, NP_]
    edge_types = [(1, 0, edge_index_p2a), (0, 1, edge_index_a2p)]
    sel = (jnp.arange(C)[:, None] // D == jnp.arange(H)[None, :]
           ).astype(jnp.float32)
    selt = sel.T
    inv_sqrt_d = 1.0 / np.sqrt(D).astype(np.float32)

    for l in range(L):
        proj = []
        for t in range(2):
            e = 1 - t
            scale = p_rel[l, e] * inv_sqrt_d
            wke = jnp.einsum('chd,hde->che', Wk[l, t].reshape(C, H, D),
                             a_rel[l, e]) * scale[None, :, None]
            bke = (jnp.einsum('hd,hde->he', bk[l, t].reshape(H, D),
                              a_rel[l, e]) * scale[:, None]).reshape(C)
            wve = jnp.einsum('chd,hde->che', Wv[l, t].reshape(C, H, D),
                             m_rel[l, e]).reshape(C, C)
            bve = jnp.einsum('hd,hde->he', bv[l, t].reshape(H, D),
                             m_rel[l, e]).reshape(C)
            wbig = jnp.concatenate([Wq[l, t], wke.reshape(C, C), wve], axis=1)
            bbig = jnp.concatenate([bq[l, t], bke, bve])
            proj.append(_dense_proj(xs[t], wbig, bbig))

        agg = [None, None]
        for e in range(2):
            st, dt, ei = edge_types[e]
            src, dst = ei[0], ei[1]
            qd = jnp.take(proj[dt][:, :C], dst, axis=0)
            kvs = jnp.take(proj[st][:, C:], src, axis=0)
            ex, msg = _edge_phase(qd, kvs, sel, selt)
            denom = jax.ops.segment_sum(ex, dst, num_segments=Ns[dt])
            unnorm = jax.ops.segment_sum(msg, dst, num_segments=Ns[dt])
            agg[dt] = unnorm / jnp.repeat(denom + 1e-16, D, axis=1)

        new_xs = []
        for t in range(2):
            s = jax.nn.sigmoid(skip[l, t])
            c = jnp.full((1, C), 1.0, jnp.float32) * (1.0 - s)
            new_xs.append(_out_proj(agg[t], Wa[l, t] * s, ba[l, t] * s,
                                    xs[t], c))
        xs = new_xs
    return (xs[0], xs[1])


# speedup vs baseline: 15.3181x; 1.1550x over previous
import functools
import numpy as np
import jax
import jax.numpy as jnp
from jax.experimental import pallas as pl

NA = 50000
NP_ = 50000
E = 500000
C = 128
H = 4
D = 32
L = 2

BLK = 2000
BLKE = 4000


def _matmul_block(x_ref, w_ref, b_ref, o_ref, *, activation):
    x = x_ref[...]
    acc = jnp.dot(x, w_ref[...], preferred_element_type=jnp.float32)
    acc = acc + b_ref[...]
    if activation == "relu":
        acc = jnp.maximum(acc, 0.0)
    o_ref[...] = acc


def _dense_proj(x, w, b, activation="none"):
    n, cin = x.shape
    cout = w.shape[1]
    return pl.pallas_call(
        functools.partial(_matmul_block, activation=activation),
        grid=(n // BLK,),
        in_specs=[
            pl.BlockSpec((BLK, cin), lambda i: (i, 0)),
            pl.BlockSpec((cin, cout), lambda i: (0, 0)),
            pl.BlockSpec((1, cout), lambda i: (0, 0)),
        ],
        out_specs=pl.BlockSpec((BLK, cout), lambda i: (i, 0)),
        out_shape=jax.ShapeDtypeStruct((n, cout), jnp.float32),
    )(x, w, b.reshape(1, cout))


def _edge_block(qd_ref, kv_ref, sel_ref, selt_ref, msgex_ref):
    qd = qd_ref[...]
    kv = kv_ref[...]
    ke = kv[:, :C]
    ve = kv[:, C:]
    ex = jnp.exp(jnp.dot(qd * ke, sel_ref[...],
                         preferred_element_type=jnp.float32))
    msgex_ref[:, :C] = ve * jnp.dot(ex, selt_ref[...],
                                    preferred_element_type=jnp.float32)
    msgex_ref[:, C:] = ex


def _edge_phase(qd, kv, sel, selt):
    e = qd.shape[0]
    return pl.pallas_call(
        _edge_block,
        grid=(e // BLKE,),
        in_specs=[
            pl.BlockSpec((BLKE, C), lambda i: (i, 0)),
            pl.BlockSpec((BLKE, 2 * C), lambda i: (i, 0)),
            pl.BlockSpec((C, H), lambda i: (0, 0)),
            pl.BlockSpec((H, C), lambda i: (0, 0)),
        ],
        out_specs=pl.BlockSpec((BLKE, C + H), lambda i: (i, 0)),
        out_shape=jax.ShapeDtypeStruct((e, C + H), jnp.float32),
    )(qd, kv, sel, selt)


def _out_block(u_ref, den_ref, selt_ref, w_ref, b_ref, x_ref, c_ref, o_ref):
    den = jnp.dot(den_ref[...], selt_ref[...],
                  preferred_element_type=jnp.float32) + 1e-16
    g = u_ref[...] / den
    acc = jnp.dot(jax.nn.gelu(g), w_ref[...],
                  preferred_element_type=jnp.float32)
    o_ref[...] = acc + b_ref[...] + c_ref[...] * x_ref[...]


def _out_proj(u, den, selt, w, b, x_old, c):
    n = u.shape[0]
    return pl.pallas_call(
        _out_block,
        grid=(n // BLK,),
        in_specs=[
            pl.BlockSpec((BLK, C), lambda i: (i, 0)),
            pl.BlockSpec((BLK, H), lambda i: (i, 0)),
            pl.BlockSpec((H, C), lambda i: (0, 0)),
            pl.BlockSpec((C, C), lambda i: (0, 0)),
            pl.BlockSpec((1, C), lambda i: (0, 0)),
            pl.BlockSpec((BLK, C), lambda i: (i, 0)),
            pl.BlockSpec((1, C), lambda i: (0, 0)),
        ],
        out_specs=pl.BlockSpec((BLK, C), lambda i: (i, 0)),
        out_shape=jax.ShapeDtypeStruct((n, C), jnp.float32),
    )(u, den, selt, w, b.reshape(1, C), x_old, c)


def kernel(x_author, x_paper, edge_index_p2a, edge_index_a2p, lin_W, lin_b,
           Wk, bk, Wq, bq, Wv, bv, Wa, ba, skip, a_rel, m_rel, p_rel):
    xs = [_dense_proj(x_author, lin_W[0], lin_b[0], "relu"),
          _dense_proj(x_paper, lin_W[1], lin_b[1], "relu")]
    Ns = [NA, NP_]
    edge_types = [(1, 0, edge_index_p2a), (0, 1, edge_index_a2p)]
    sel = (jnp.arange(C)[:, None] // D == jnp.arange(H)[None, :]
           ).astype(jnp.float32)
    selt = sel.T
    inv_sqrt_d = 1.0 / np.sqrt(D).astype(np.float32)

    for l in range(L):
        proj = []
        for t in range(2):
            e = 1 - t
            scale = p_rel[l, e] * inv_sqrt_d
            wke = jnp.einsum('chd,hde->che', Wk[l, t].reshape(C, H, D),
                             a_rel[l, e]) * scale[None, :, None]
            bke = (jnp.einsum('hd,hde->he', bk[l, t].reshape(H, D),
                              a_rel[l, e]) * scale[:, None]).reshape(C)
            wve = jnp.einsum('chd,hde->che', Wv[l, t].reshape(C, H, D),
                             m_rel[l, e]).reshape(C, C)
            bve = jnp.einsum('hd,hde->he', bv[l, t].reshape(H, D),
                             m_rel[l, e]).reshape(C)
            wbig = jnp.concatenate([Wq[l, t], wke.reshape(C, C), wve], axis=1)
            bbig = jnp.concatenate([bq[l, t], bke, bve])
            proj.append(_dense_proj(xs[t], wbig, bbig))

        agg = [None, None]
        for e in range(2):
            st, dt, ei = edge_types[e]
            src, dst = ei[0], ei[1]
            qd = jnp.take(proj[dt][:, :C], dst, axis=0)
            kvs = jnp.take(proj[st][:, C:], src, axis=0)
            msgex = _edge_phase(qd, kvs, sel, selt)
            tot = jax.ops.segment_sum(msgex, dst, num_segments=Ns[dt])
            agg[dt] = (tot[:, :C], tot[:, C:])

        new_xs = []
        for t in range(2):
            s = jax.nn.sigmoid(skip[l, t])
            c = jnp.full((1, C), 1.0, jnp.float32) * (1.0 - s)
            unnorm, denom = agg[t]
            new_xs.append(_out_proj(unnorm, denom, selt,
                                    Wa[l, t] * s, ba[l, t] * s, xs[t], c))
        xs = new_xs
    return (xs[0], xs[1])
